# Initial kernel scaffold; baseline (speedup 1.0000x reference)
import jax
import jax.numpy as jnp
from jax.experimental import pallas as pl


def kernel(h, edge_features, edge_ft_upper, edge_index, edge_index_upper, Wn1, We1, a1, Wn2, We2, a2):
    raise NotImplementedError("write your pallas kernel here")


# speedup vs baseline: 2.0028x; 2.0028x over previous
import functools
import jax
import jax.numpy as jnp
from jax.experimental import pallas as pl


def _mm_kernel(x_ref, w_ref, o_ref):
    o_ref[...] = jnp.dot(x_ref[...], w_ref[...],
                         preferred_element_type=jnp.float32)


def _mm(x, w, bm=1024):
    M, K = x.shape
    _, N = w.shape
    grid = (pl.cdiv(M, bm),)
    return pl.pallas_call(
        _mm_kernel,
        grid=grid,
        in_specs=[
            pl.BlockSpec((bm, K), lambda i: (i, 0)),
            pl.BlockSpec((K, N), lambda i: (0, 0)),
        ],
        out_specs=pl.BlockSpec((bm, N), lambda i: (i, 0)),
        out_shape=jax.ShapeDtypeStruct((M, N), jnp.float32),
    )(x, w)


def _egat_v0(nfeat, efeat, src, dst, Wn, We, a, n_dst):
    H, D = a.shape
    z = _mm(nfeat, Wn).reshape(-1, H, D)
    ez = _mm(efeat, We).reshape(-1, H, D)
    f = z[src] + z[dst] + ez
    logits = jnp.sum(jax.nn.leaky_relu(f, negative_slope=0.2) * a[None], axis=-1)
    ex = jnp.exp(logits)
    denom = jax.ops.segment_sum(ex, dst, num_segments=n_dst)
    num = jax.ops.segment_sum(z[src] * ex[..., None], dst, num_segments=n_dst)
    hout = num / (denom[..., None] + 1e-9)
    return jax.nn.elu(hout)


def kernel(h, edge_features, edge_ft_upper, edge_index, edge_index_upper,
           Wn1, We1, a1, Wn2, We2, a2):
    src1, dst1 = edge_index[0], edge_index[1]
    node_out = _egat_v0(h, edge_features, src1, dst1, Wn1, We1, a1, h.shape[0])
    node_embeddings = node_out.reshape(node_out.shape[0], -1)
    src2, dst2 = edge_index_upper[0], edge_index_upper[1]
    e_out = _egat_v0(edge_features, edge_ft_upper, src2, dst2, Wn2, We2, a2,
                     edge_features.shape[0])
    edge_embeddings = e_out.reshape(e_out.shape[0], -1)
    return (node_embeddings, edge_embeddings)
